# Initial kernel scaffold; baseline (speedup 1.0000x reference)
import jax
import jax.numpy as jnp
from jax.experimental import pallas as pl


def kernel(x, gate_w, gate_b, W_in, conv_w, conv_b, dt_bias, A_log, D, norm_w, W_out):
    raise NotImplementedError("write your pallas kernel here")


# speedup vs baseline: 31.6082x; 31.6082x over previous
import jax
import jax.numpy as jnp
from jax.experimental import pallas as pl
import jax.experimental.pallas.tpu as pltpu

_D_MODEL = 768
_D_STATE = 128
_NUM_EXPERTS = 8
_D_CONV = 4
_D_INNER = 1536
_NHEADS = 24
_HEADDIM = 64
_CONV_DIM = _D_INNER + 2 * _D_STATE
_D_IN_PROJ = 2 * _D_INNER + 2 * _D_STATE + _NHEADS
_Q = 256


def _softplus(x):
    return jnp.maximum(x, 0.0) + jnp.log1p(jnp.exp(-jnp.abs(x)))


def _silu(x):
    return x * jax.nn.sigmoid(x)


def _dot(a, b, dims):
    return jax.lax.dot_general(a, b, (dims, ((), ())),
                               preferred_element_type=jnp.float32)


def _gate_body(x_ref, gate_w_ref, gate_b_ref, noise_ref, mask_ref):
    xb = x_ref[0]
    logits = _dot(xb, gate_w_ref[...], ((1,), (1,)))
    logits = logits + gate_b_ref[...] + noise_ref[0]
    m = logits - jnp.max(logits, axis=1, keepdims=True)
    p = jnp.exp(m)
    p = p / jnp.sum(p, axis=1, keepdims=True)
    iota = jax.lax.broadcasted_iota(jnp.int32, p.shape, 1)
    v0 = jnp.max(p, axis=1, keepdims=True)
    idx0 = jnp.min(jnp.where(p == v0, iota, _NUM_EXPERTS), axis=1,
                   keepdims=True)
    m0 = iota == idx0
    p1 = jnp.where(m0, -jnp.inf, p)
    v1 = jnp.max(p1, axis=1, keepdims=True)
    idx1 = jnp.min(jnp.where(p1 == v1, iota, _NUM_EXPERTS), axis=1,
                   keepdims=True)
    m1 = iota == idx1
    mask_ref[0] = (jnp.where(m0, v0, 0.0) + jnp.where(m1, v1, 0.0)) / (v0 + v1)


def _moe_body(x_ref, mask_ref, W_in_ref, conv_w_ref, conv_b_ref,
              dt_bias_ref, A_log_ref, D_ref, norm_w_ref, W_out_ref,
              out_ref, state_ref, tail_ref, ybuf_ref):
    e = pl.program_id(0)
    b = pl.program_id(1)
    c = pl.program_id(2)
    Q = _Q

    @pl.when((e == 0) & (b == 0) & (c == 0))
    def _init():
        out_ref[...] = jnp.zeros_like(out_ref)

    @pl.when(c == 0)
    def _reset():
        state_ref[...] = jnp.zeros_like(state_ref)
        tail_ref[...] = jnp.zeros_like(tail_ref)

    xc = x_ref[0]
    zxbcdt = _dot(xc, W_in_ref[0], ((1,), (1,)))
    z = zxbcdt[:, :_D_INNER]
    xBC_raw = zxbcdt[:, _D_INNER:_D_INNER + _CONV_DIM]
    dtr = zxbcdt[:, _D_INNER + _CONV_DIM:]

    dt = _softplus(dtr + dt_bias_ref[0])
    A = -jnp.exp(A_log_ref[0])
    a = dt * A

    r = jax.lax.broadcasted_iota(jnp.int32, (Q, Q), 0)
    s = jax.lax.broadcasted_iota(jnp.int32, (Q, Q), 1)
    tri_b = s <= r
    tri = tri_b.astype(jnp.float32)
    cum = _dot(tri, a, ((1,), (0,)))
    cumT = cum.T

    xp = jnp.concatenate([tail_ref[:_D_CONV - 1], xBC_raw], axis=0)
    cw = conv_w_ref[0]
    acc = jnp.broadcast_to(conv_b_ref[0], (Q, _CONV_DIM))
    for i in range(_D_CONV):
        acc = acc + xp[i:i + Q, :] * cw[i:i + 1, :]
    xBC = _silu(acc)
    tail_ref[:_D_CONV - 1] = xBC_raw[Q - (_D_CONV - 1):, :]

    Bc = xBC[:, _D_INNER:_D_INNER + _D_STATE]
    Cc = xBC[:, _D_INNER + _D_STATE:]
    G = _dot(Cc, Bc, ((1,), (1,)))

    for h in range(_NHEADS):
        lo = h * _HEADDIM
        xs_h = xBC[:, lo:lo + _HEADDIM]
        dt_h = dt[:, h:h + 1]
        cum_col = cum[:, h:h + 1]
        cum_row = cumT[h:h + 1, :]
        M = jnp.where(tri_b, jnp.exp(cum_col - cum_row), 0.0) * G
        xdt_h = xs_h * dt_h
        y_h = _dot(M, xdt_h, ((1,), (0,)))
        S_h = state_ref[h]
        y_h = y_h + jnp.exp(cum_col) * _dot(Cc, S_h, ((1,), (1,)))
        y_h = y_h + xs_h * D_ref[0, :, h:h + 1]
        ybuf_ref[:, lo:lo + _HEADDIM] = y_h
        cum_end = cum[Q - 1:Q, h:h + 1]
        xw = xdt_h * jnp.exp(cum_end - cum_col)
        state_ref[h] = jnp.exp(cum_end) * S_h + _dot(xw, Bc, ((0,), (0,)))

    y = ybuf_ref[...]
    y = y * _silu(z)
    ms = jnp.mean(y * y, axis=1, keepdims=True)
    y = y * jax.lax.rsqrt(ms + 1e-5) * norm_w_ref[0]

    mrow = mask_ref[b, pl.ds(c * Q, Q), :]
    iota_e = jax.lax.broadcasted_iota(jnp.int32, mrow.shape, 1)
    mcol = jnp.sum(jnp.where(iota_e == e, mrow, 0.0), axis=1, keepdims=True)
    y = y * mcol
    outc = _dot(y, W_out_ref[0], ((1,), (1,)))
    out_ref[b, pl.ds(c * Q, Q), :] += outc


def kernel(x, gate_w, gate_b, W_in, conv_w, conv_b, dt_bias, A_log, D,
           norm_w, W_out):
    B, L, _ = x.shape
    E = _NUM_EXPERTS
    noise = jax.random.normal(jax.random.key(42), (B, L, E),
                              dtype=jnp.float32) * 0.01
    conv_w_t = conv_w.transpose(0, 2, 1)
    gate_b2 = gate_b.reshape(1, E)
    conv_b3 = conv_b.reshape(E, 1, _CONV_DIM)
    dt_bias3 = dt_bias.reshape(E, 1, _NHEADS)
    A_log3 = A_log.reshape(E, 1, _NHEADS)
    D3 = D.reshape(E, 1, _NHEADS)
    norm_w3 = norm_w.reshape(E, 1, _D_INNER)

    mask = pl.pallas_call(
        _gate_body,
        grid=(B,),
        in_specs=[
            pl.BlockSpec((1, L, _D_MODEL), lambda b: (b, 0, 0)),
            pl.BlockSpec((E, _D_MODEL), lambda b: (0, 0)),
            pl.BlockSpec((1, E), lambda b: (0, 0)),
            pl.BlockSpec((1, L, E), lambda b: (b, 0, 0)),
        ],
        out_specs=pl.BlockSpec((1, L, E), lambda b: (b, 0, 0)),
        out_shape=jax.ShapeDtypeStruct((B, L, E), jnp.float32),
    )(x, gate_w, gate_b2, noise)

    nc = L // _Q
    grid = (E, B, nc)
    expert3 = lambda e, b, c: (e, 0, 0)
    out = pl.pallas_call(
        _moe_body,
        grid=grid,
        in_specs=[
            pl.BlockSpec((1, _Q, _D_MODEL), lambda e, b, c: (b, c, 0)),
            pl.BlockSpec((B, L, E), lambda e, b, c: (0, 0, 0)),
            pl.BlockSpec((1, _D_IN_PROJ, _D_MODEL), expert3),
            pl.BlockSpec((1, _D_CONV, _CONV_DIM), expert3),
            pl.BlockSpec((1, 1, _CONV_DIM), expert3),
            pl.BlockSpec((1, 1, _NHEADS), expert3),
            pl.BlockSpec((1, 1, _NHEADS), expert3),
            pl.BlockSpec((1, 1, _NHEADS), expert3),
            pl.BlockSpec((1, 1, _D_INNER), expert3),
            pl.BlockSpec((1, _D_MODEL, _D_INNER), expert3),
        ],
        out_specs=pl.BlockSpec((B, L, _D_MODEL), lambda e, b, c: (0, 0, 0)),
        out_shape=jax.ShapeDtypeStruct((B, L, _D_MODEL), jnp.float32),
        scratch_shapes=[
            pltpu.VMEM((_NHEADS, _HEADDIM, _D_STATE), jnp.float32),
            pltpu.VMEM((8, _CONV_DIM), jnp.float32),
            pltpu.VMEM((_Q, _D_INNER), jnp.float32),
        ],
    )(x, mask, W_in, conv_w_t, conv_b3, dt_bias3, A_log3, D3, norm_w3,
      W_out)
    return out


# speedup vs baseline: 32.6048x; 1.0315x over previous
import jax
import jax.numpy as jnp
from jax.experimental import pallas as pl
import jax.experimental.pallas.tpu as pltpu

_D_MODEL = 768
_D_STATE = 128
_NUM_EXPERTS = 8
_D_CONV = 4
_D_INNER = 1536
_NHEADS = 24
_HEADDIM = 64
_CONV_DIM = _D_INNER + 2 * _D_STATE
_D_IN_PROJ = 2 * _D_INNER + 2 * _D_STATE + _NHEADS
_Q = 256


def _softplus(x):
    return jnp.maximum(x, 0.0) + jnp.log1p(jnp.exp(-jnp.abs(x)))


def _silu(x):
    return x * jax.nn.sigmoid(x)


def _dot(a, b, dims):
    return jax.lax.dot_general(a, b, (dims, ((), ())),
                               preferred_element_type=jnp.float32)


def _xdot(a, b, dims):
    return jax.lax.dot_general(a, b, (dims, ((), ())),
                               preferred_element_type=jnp.float32,
                               precision=jax.lax.Precision.HIGHEST)


def _gate_body(x_ref, gate_w_ref, gate_b_ref, noise_ref, mask_ref):
    xb = x_ref[0]
    logits = _dot(xb, gate_w_ref[...], ((1,), (1,)))
    logits = logits + gate_b_ref[...] + noise_ref[0]
    m = logits - jnp.max(logits, axis=1, keepdims=True)
    p = jnp.exp(m)
    p = p / jnp.sum(p, axis=1, keepdims=True)
    iota = jax.lax.broadcasted_iota(jnp.int32, p.shape, 1)
    v0 = jnp.max(p, axis=1, keepdims=True)
    idx0 = jnp.min(jnp.where(p == v0, iota, _NUM_EXPERTS), axis=1,
                   keepdims=True)
    m0 = iota == idx0
    p1 = jnp.where(m0, -jnp.inf, p)
    v1 = jnp.max(p1, axis=1, keepdims=True)
    idx1 = jnp.min(jnp.where(p1 == v1, iota, _NUM_EXPERTS), axis=1,
                   keepdims=True)
    m1 = iota == idx1
    mask_ref[0] = (jnp.where(m0, v0, 0.0) + jnp.where(m1, v1, 0.0)) / (v0 + v1)


def _moe_body(x_ref, mask_ref, W_in_ref, conv_w_ref, conv_b_ref,
              dt_bias_ref, A_log_ref, D_ref, norm_w_ref, W_out_ref,
              out_ref, state_ref, tail_ref, ybuf1_ref, ybuf2_ref):
    e = pl.program_id(0)
    b = pl.program_id(1)
    c = pl.program_id(2)
    Q = _Q
    H = _NHEADS
    P = _HEADDIM

    @pl.when((e == 0) & (b == 0) & (c == 0))
    def _init():
        out_ref[...] = jnp.zeros_like(out_ref)

    @pl.when(c == 0)
    def _reset():
        state_ref[...] = jnp.zeros_like(state_ref)
        tail_ref[...] = jnp.zeros_like(tail_ref)

    ih = jax.lax.broadcasted_iota(jnp.int32, (H, _D_INNER), 0)
    ij = jax.lax.broadcasted_iota(jnp.int32, (H, _D_INNER), 1) // P
    Em = (ih == ij).astype(jnp.float32)

    xc = x_ref[0]
    zxbcdt = _dot(xc, W_in_ref[0], ((1,), (1,)))
    z = zxbcdt[:, :_D_INNER]
    xBC_raw = zxbcdt[:, _D_INNER:_D_INNER + _CONV_DIM]

    W_dt = W_in_ref[0][_D_INNER + _CONV_DIM:, :]
    dtT_raw = _dot(W_dt, xc, ((1,), (1,)))
    dtT = _softplus(dtT_raw + dt_bias_ref[0])
    aT = dtT * (-jnp.exp(A_log_ref[0]))

    r = jax.lax.broadcasted_iota(jnp.int32, (Q, Q), 0)
    s = jax.lax.broadcasted_iota(jnp.int32, (Q, Q), 1)
    tri_b = s <= r
    triu = (r <= s).astype(jnp.float32)
    cumT = _xdot(aT, triu, ((1,), (0,)))
    EcumT = jnp.exp(cumT)
    cum_endT = cumT[:, Q - 1:Q]
    wdecT = jnp.exp(cum_endT - cumT)
    sdecT = jnp.exp(cum_endT)

    cum_full = _xdot(cumT, Em, ((0,), (0,)))
    Ecum_full = _xdot(EcumT, Em, ((0,), (0,)))
    wdec_full = _xdot(wdecT, Em, ((0,), (0,)))
    dt_full = _xdot(dtT, Em, ((0,), (0,)))
    D_row = _xdot(D_ref[0], Em, ((1,), (0,)))

    xp = jnp.concatenate([tail_ref[:_D_CONV - 1], xBC_raw], axis=0)
    cw = conv_w_ref[0]
    acc = jnp.broadcast_to(conv_b_ref[0], (Q, _CONV_DIM))
    for i in range(_D_CONV):
        acc = acc + xp[i:i + Q, :] * cw[i:i + 1, :]
    xBC = _silu(acc)
    tail_ref[:_D_CONV - 1] = xBC_raw[Q - (_D_CONV - 1):, :]

    xs_full = xBC[:, :_D_INNER]
    Bc16 = xBC[:, _D_INNER:_D_INNER + _D_STATE].astype(jnp.bfloat16)
    Cc16 = xBC[:, _D_INNER + _D_STATE:].astype(jnp.bfloat16)
    G = _dot(Cc16, Bc16, ((1,), (1,)))

    xdt_full = xs_full * dt_full
    xdt16 = xdt_full.astype(jnp.bfloat16)
    xw16 = (xdt_full * wdec_full).astype(jnp.bfloat16)

    for h in range(H):
        lo = h * P
        cum_col = cum_full[:, lo:lo + 1]
        cum_row = cumT[h:h + 1, :]
        M16 = (jnp.where(tri_b, jnp.exp(cum_col - cum_row), 0.0)
               * G).astype(jnp.bfloat16)
        ybuf1_ref[:, lo:lo + P] = _dot(M16, xdt16[:, lo:lo + P], ((1,), (0,)))
        S_h = state_ref[h]
        ybuf2_ref[:, lo:lo + P] = _dot(Cc16, S_h.astype(jnp.bfloat16),
                                       ((1,), (1,)))
        state_ref[h] = (sdecT[h:h + 1, :] * S_h
                        + _dot(xw16[:, lo:lo + P], Bc16, ((0,), (0,))))

    y = ybuf1_ref[...] + Ecum_full * ybuf2_ref[...] + xs_full * D_row
    y = y * _silu(z)
    ms = jnp.mean(y * y, axis=1, keepdims=True)
    y = y * jax.lax.rsqrt(ms + 1e-5) * norm_w_ref[0]

    mrow = mask_ref[b, pl.ds(c * Q, Q), :]
    iota_e = jax.lax.broadcasted_iota(jnp.int32, mrow.shape, 1)
    mcol = jnp.sum(jnp.where(iota_e == e, mrow, 0.0), axis=1, keepdims=True)
    y16 = (y * mcol).astype(jnp.bfloat16)
    outc = _dot(y16, W_out_ref[0], ((1,), (1,)))
    out_ref[b, pl.ds(c * Q, Q), :] += outc


def kernel(x, gate_w, gate_b, W_in, conv_w, conv_b, dt_bias, A_log, D,
           norm_w, W_out):
    B, L, _ = x.shape
    E = _NUM_EXPERTS
    noise = jax.random.normal(jax.random.key(42), (B, L, E),
                              dtype=jnp.float32) * 0.01
    x16 = x.astype(jnp.bfloat16)
    W_in16 = W_in.astype(jnp.bfloat16)
    W_out16 = W_out.astype(jnp.bfloat16)
    conv_w_t = conv_w.transpose(0, 2, 1)
    gate_b2 = gate_b.reshape(1, E)
    conv_b3 = conv_b.reshape(E, 1, _CONV_DIM)
    dt_biasT = dt_bias.reshape(E, _NHEADS, 1)
    A_logT = A_log.reshape(E, _NHEADS, 1)
    D3 = D.reshape(E, 1, _NHEADS)
    norm_w3 = norm_w.reshape(E, 1, _D_INNER)

    mask = pl.pallas_call(
        _gate_body,
        grid=(B,),
        in_specs=[
            pl.BlockSpec((1, L, _D_MODEL), lambda b: (b, 0, 0)),
            pl.BlockSpec((E, _D_MODEL), lambda b: (0, 0)),
            pl.BlockSpec((1, E), lambda b: (0, 0)),
            pl.BlockSpec((1, L, E), lambda b: (b, 0, 0)),
        ],
        out_specs=pl.BlockSpec((1, L, E), lambda b: (b, 0, 0)),
        out_shape=jax.ShapeDtypeStruct((B, L, E), jnp.float32),
    )(x, gate_w, gate_b2, noise)

    nc = L // _Q
    grid = (E, B, nc)
    expert3 = lambda e, b, c: (e, 0, 0)
    out = pl.pallas_call(
        _moe_body,
        grid=grid,
        in_specs=[
            pl.BlockSpec((1, _Q, _D_MODEL), lambda e, b, c: (b, c, 0)),
            pl.BlockSpec((B, L, E), lambda e, b, c: (0, 0, 0)),
            pl.BlockSpec((1, _D_IN_PROJ, _D_MODEL), expert3),
            pl.BlockSpec((1, _D_CONV, _CONV_DIM), expert3),
            pl.BlockSpec((1, 1, _CONV_DIM), expert3),
            pl.BlockSpec((1, _NHEADS, 1), expert3),
            pl.BlockSpec((1, _NHEADS, 1), expert3),
            pl.BlockSpec((1, 1, _NHEADS), expert3),
            pl.BlockSpec((1, 1, _D_INNER), expert3),
            pl.BlockSpec((1, _D_MODEL, _D_INNER), expert3),
        ],
        out_specs=pl.BlockSpec((B, L, _D_MODEL), lambda e, b, c: (0, 0, 0)),
        out_shape=jax.ShapeDtypeStruct((B, L, _D_MODEL), jnp.float32),
        scratch_shapes=[
            pltpu.VMEM((_NHEADS, _HEADDIM, _D_STATE), jnp.float32),
            pltpu.VMEM((8, _CONV_DIM), jnp.float32),
            pltpu.VMEM((_Q, _D_INNER), jnp.float32),
            pltpu.VMEM((_Q, _D_INNER), jnp.float32),
        ],
    )(x16, mask, W_in16, conv_w_t, conv_b3, dt_biasT, A_logT, D3,
      norm_w3, W_out16)
    return out


# speedup vs baseline: 53.2013x; 1.6317x over previous
import functools

import jax
import jax.numpy as jnp
from jax.experimental import pallas as pl
import jax.experimental.pallas.tpu as pltpu
from jax.experimental.pallas import tpu_sc as plsc

_D_MODEL = 768
_D_STATE = 128
_NUM_EXPERTS = 8
_D_CONV = 4
_D_INNER = 1536
_NHEADS = 24
_HEADDIM = 64
_CONV_DIM = _D_INNER + 2 * _D_STATE
_D_IN_PROJ = 2 * _D_INNER + 2 * _D_STATE + _NHEADS
_Q = 256
_LANES = 16


def _softplus(x):
    return jnp.maximum(x, 0.0) + jnp.log1p(jnp.exp(-jnp.abs(x)))


def _silu(x):
    return x * jax.nn.sigmoid(x)


def _dot(a, b, dims):
    return jax.lax.dot_general(a, b, (dims, ((), ())),
                               preferred_element_type=jnp.float32)


def _xdot(a, b, dims):
    return jax.lax.dot_general(a, b, (dims, ((), ())),
                               preferred_element_type=jnp.float32,
                               precision=jax.lax.Precision.HIGHEST)


def _logits_body(x_ref, gate_w_ref, gate_b_ref, noise_ref, out_ref):
    xb = x_ref[0]
    lg = _dot(gate_w_ref[...], xb, ((1,), (1,)))
    out_ref[0] = lg + gate_b_ref[...] + noise_ref[0]


def _sc_gate_body(lg_hbm, mask_hbm, slab, oslab):
    E = _NUM_EXPERTS
    info = plsc.get_sparse_core_info()
    nc, ns = info.num_cores, info.num_subcores
    t_per_w = lg_hbm.shape[1] // (nc * ns)
    wid = jax.lax.axis_index("s") * nc + jax.lax.axis_index("c")
    base = wid * t_per_w
    for e in range(E):
        pltpu.sync_copy(lg_hbm.at[e, pl.ds(base, t_per_w)], slab.at[e])
    for j in range(t_per_w // _LANES):
        sl = pl.ds(j * _LANES, _LANES)
        lgs = [slab[e, sl] for e in range(E)]
        m = lgs[0]
        for e in range(1, E):
            m = jnp.maximum(m, lgs[e])
        es = [jnp.exp(v - m) for v in lgs]
        s = es[0]
        for e in range(1, E):
            s = s + es[e]
        ps = [v / s for v in es]
        v0 = ps[0]
        for e in range(1, E):
            v0 = jnp.maximum(v0, ps[e])
        seen = jnp.zeros_like(v0)
        sel0 = []
        for e in range(E):
            hit = jnp.where(ps[e] == v0, 1.0 - seen, 0.0)
            sel0.append(hit)
            seen = seen + hit
        p1s = [jnp.where(sel0[e] > 0.0, -jnp.inf, ps[e]) for e in range(E)]
        v1 = p1s[0]
        for e in range(1, E):
            v1 = jnp.maximum(v1, p1s[e])
        seen1 = jnp.zeros_like(v0)
        den = v0 + v1
        for e in range(E):
            hit1 = jnp.where(p1s[e] == v1, 1.0 - seen1, 0.0)
            seen1 = seen1 + hit1
            oslab[e, sl] = (sel0[e] * v0 + hit1 * v1) / den
    for e in range(E):
        pltpu.sync_copy(oslab.at[e], mask_hbm.at[e, pl.ds(base, t_per_w)])


def _moe_body(x_ref, mask_ref, W_in_ref, conv_w_ref, conv_b_ref,
              dt_bias_ref, A_log_ref, D_ref, norm_w_ref, W_out_ref,
              out_ref, state_ref, tail_ref, ybuf1_ref):
    e = pl.program_id(0)
    b = pl.program_id(1)
    c = pl.program_id(2)
    Q = _Q
    H = _NHEADS
    P = _HEADDIM

    @pl.when((e == 0) & (b == 0) & (c == 0))
    def _init():
        out_ref[...] = jnp.zeros_like(out_ref)

    @pl.when(c == 0)
    def _reset():
        state_ref[...] = jnp.zeros_like(state_ref)
        tail_ref[...] = jnp.zeros_like(tail_ref)

    ih = jax.lax.broadcasted_iota(jnp.int32, (H, _D_INNER), 0)
    ij = jax.lax.broadcasted_iota(jnp.int32, (H, _D_INNER), 1) // P
    Em = (ih == ij).astype(jnp.float32)
    ih2 = jax.lax.broadcasted_iota(jnp.int32, (_D_INNER, H), 0) // P
    ij2 = jax.lax.broadcasted_iota(jnp.int32, (_D_INNER, H), 1)
    Em2 = (ih2 == ij2).astype(jnp.float32)

    xc = x_ref[0]
    zxbcdt = _dot(xc, W_in_ref[0], ((1,), (1,)))
    z = zxbcdt[:, :_D_INNER]
    xBC_raw = zxbcdt[:, _D_INNER:_D_INNER + _CONV_DIM]

    W_dt = W_in_ref[0][_D_INNER + _CONV_DIM:, :]
    dtT_raw = _dot(W_dt, xc, ((1,), (1,)))
    dtT = _softplus(dtT_raw + dt_bias_ref[0])
    aT = dtT * (-jnp.exp(A_log_ref[0]))

    r = jax.lax.broadcasted_iota(jnp.int32, (Q, Q), 0)
    s = jax.lax.broadcasted_iota(jnp.int32, (Q, Q), 1)
    tri_b = s <= r
    triu = (r <= s).astype(jnp.float32)
    cumT = _xdot(aT, triu, ((1,), (0,)))
    cum = _xdot(triu, aT, ((0,), (1,)))
    EcumT = jnp.exp(cumT)
    cum_endT = cumT[:, Q - 1:Q]
    wdecT = jnp.exp(cum_endT - cumT)
    sdecT = jnp.exp(cum_endT)

    Ecum_full = _dot(EcumT, Em, ((0,), (0,)))
    wdec_full = _dot(wdecT, Em, ((0,), (0,)))
    dt_full = _dot(dtT, Em, ((0,), (0,)))
    D_row = _dot(D_ref[0], Em, ((1,), (0,)))
    sdec_col = _dot(Em2, sdecT, ((1,), (0,)))

    xp = jnp.concatenate([tail_ref[:_D_CONV - 1], xBC_raw], axis=0)
    cw = conv_w_ref[0]
    acc = jnp.broadcast_to(conv_b_ref[0], (Q, _CONV_DIM))
    for i in range(_D_CONV):
        acc = acc + xp[i:i + Q, :] * cw[i:i + 1, :]
    xBC = _silu(acc)
    tail_ref[:_D_CONV - 1] = xBC_raw[Q - (_D_CONV - 1):, :]

    xs_full = xBC[:, :_D_INNER]
    Bc16 = xBC[:, _D_INNER:_D_INNER + _D_STATE].astype(jnp.bfloat16)
    Cc16 = xBC[:, _D_INNER + _D_STATE:].astype(jnp.bfloat16)
    G = _dot(Cc16, Bc16, ((1,), (1,)))

    xdt_full = xs_full * dt_full
    xdt16 = xdt_full.astype(jnp.bfloat16)
    xw16 = (xdt_full * wdec_full).astype(jnp.bfloat16)
    G16 = G.astype(jnp.bfloat16)

    for h in range(H):
        lo = h * P
        cum_col = cum[:, h:h + 1]
        cum_row = cumT[h:h + 1, :]
        M16 = (jnp.where(tri_b, jnp.exp(cum_col - cum_row), 0.0)
               .astype(jnp.bfloat16) * G16)
        ybuf1_ref[:, lo:lo + P] = _dot(M16, xdt16[:, lo:lo + P], ((1,), (0,)))

    S = state_ref[...]
    inter_full = _dot(Cc16, S.astype(jnp.bfloat16), ((1,), (1,)))
    state_ref[...] = sdec_col * S + _dot(xw16, Bc16, ((0,), (0,)))

    y = ybuf1_ref[...] + Ecum_full * inter_full + xs_full * D_row
    y = y * _silu(z)
    ms = jnp.mean(y * y, axis=1, keepdims=True)
    y = y * jax.lax.rsqrt(ms + 1e-5) * norm_w_ref[0]

    mcol = mask_ref[0, b, pl.ds(c * Q, Q)].reshape(Q, 1)
    y16 = (y * mcol).astype(jnp.bfloat16)
    outc = _dot(y16, W_out_ref[0], ((1,), (1,)))
    out_ref[b, pl.ds(c * Q, Q), :] += outc


def kernel(x, gate_w, gate_b, W_in, conv_w, conv_b, dt_bias, A_log, D,
           norm_w, W_out):
    B, L, _ = x.shape
    E = _NUM_EXPERTS
    T = B * L
    noise = jax.random.normal(jax.random.key(42), (B, L, E),
                              dtype=jnp.float32) * 0.01
    noiseT = noise.transpose(0, 2, 1)
    x16 = x.astype(jnp.bfloat16)
    W_in16 = W_in.astype(jnp.bfloat16)
    W_out16 = W_out.astype(jnp.bfloat16)
    conv_w_t = conv_w.transpose(0, 2, 1)
    gate_b2 = gate_b.reshape(E, 1)
    conv_b3 = conv_b.reshape(E, 1, _CONV_DIM)
    dt_biasT = dt_bias.reshape(E, _NHEADS, 1)
    A_logT = A_log.reshape(E, _NHEADS, 1)
    D3 = D.reshape(E, 1, _NHEADS)
    norm_w3 = norm_w.reshape(E, 1, _D_INNER)

    logits = pl.pallas_call(
        _logits_body,
        grid=(B,),
        in_specs=[
            pl.BlockSpec((1, L, _D_MODEL), lambda b: (b, 0, 0)),
            pl.BlockSpec((E, _D_MODEL), lambda b: (0, 0)),
            pl.BlockSpec((E, 1), lambda b: (0, 0)),
            pl.BlockSpec((1, E, L), lambda b: (b, 0, 0)),
        ],
        out_specs=pl.BlockSpec((1, E, L), lambda b: (b, 0, 0)),
        out_shape=jax.ShapeDtypeStruct((B, E, L), jnp.float32),
    )(x, gate_w, gate_b2, noiseT)
    logitsT = logits.transpose(1, 0, 2).reshape(E, T)

    info = plsc.get_sparse_core_info()
    t_per_w = T // (info.num_cores * info.num_subcores)
    sc_gate = functools.partial(
        pl.kernel,
        out_type=jax.ShapeDtypeStruct((E, T), jnp.float32),
        mesh=plsc.VectorSubcoreMesh(core_axis_name="c", subcore_axis_name="s"),
        scratch_types=[
            pltpu.VMEM((E, t_per_w), jnp.float32),
            pltpu.VMEM((E, t_per_w), jnp.float32),
        ],
    )(_sc_gate_body)
    maskT = sc_gate(logitsT).reshape(E, B, L)

    nc = L // _Q
    grid = (E, B, nc)
    expert3 = lambda e, b, c: (e, 0, 0)
    out = pl.pallas_call(
        _moe_body,
        grid=grid,
        in_specs=[
            pl.BlockSpec((1, _Q, _D_MODEL), lambda e, b, c: (b, c, 0)),
            pl.BlockSpec((1, B, L), lambda e, b, c: (e, 0, 0)),
            pl.BlockSpec((1, _D_IN_PROJ, _D_MODEL), expert3),
            pl.BlockSpec((1, _D_CONV, _CONV_DIM), expert3),
            pl.BlockSpec((1, 1, _CONV_DIM), expert3),
            pl.BlockSpec((1, _NHEADS, 1), expert3),
            pl.BlockSpec((1, _NHEADS, 1), expert3),
            pl.BlockSpec((1, 1, _NHEADS), expert3),
            pl.BlockSpec((1, 1, _D_INNER), expert3),
            pl.BlockSpec((1, _D_MODEL, _D_INNER), expert3),
        ],
        out_specs=pl.BlockSpec((B, L, _D_MODEL), lambda e, b, c: (0, 0, 0)),
        out_shape=jax.ShapeDtypeStruct((B, L, _D_MODEL), jnp.float32),
        scratch_shapes=[
            pltpu.VMEM((_D_INNER, _D_STATE), jnp.float32),
            pltpu.VMEM((8, _CONV_DIM), jnp.float32),
            pltpu.VMEM((_Q, _D_INNER), jnp.float32),
        ],
    )(x16, maskT, W_in16, conv_w_t, conv_b3, dt_biasT, A_logT, D3,
      norm_w3, W_out16)
    return out
